# baseline (speedup 1.0000x reference)
import functools

import jax
import jax.numpy as jnp
from jax import lax
from jax.experimental import pallas as pl
from jax.experimental.pallas import tpu as pltpu
from jax.experimental.pallas import tpu_sc as plsc

_EMB = 32
_NC = 2
_NS = 16
_NW = _NC * _NS
_IDX_MINOR = 1280
_K_PER = 1
_CHUNK = _K_PER * _IDX_MINOR


@functools.lru_cache(maxsize=None)
def _make_gather(n_tokens: int):
    b_per_w = n_tokens // _NW
    n_chunks = b_per_w // _CHUNK
    assert b_per_w * _NW == n_tokens and n_chunks * _CHUNK == b_per_w
    assert n_chunks % 2 == 0
    idx_rows_per_w = b_per_w // _IDX_MINOR
    mesh = plsc.VectorSubcoreMesh(core_axis_name="c", subcore_axis_name="s")

    @functools.partial(
        pl.kernel,
        out_type=jax.ShapeDtypeStruct((n_tokens, _EMB), jnp.float32),
        mesh=mesh,
        compiler_params=pltpu.CompilerParams(use_tc_tiling_on_sc=False),
        scratch_types=[
            pltpu.VMEM((2, _K_PER, _IDX_MINOR), jnp.int32),
            pltpu.VMEM((2, _CHUNK, _EMB), jnp.float32),
            pltpu.SemaphoreType.DMA,
            pltpu.SemaphoreType.DMA,
            pltpu.SemaphoreType.DMA,
            pltpu.SemaphoreType.DMA,
        ],
    )
    def body(idx_hbm, table_hbm, out_hbm, idx_v, rows_v, g0, g1, o0, o1):
        wid = lax.axis_index("s") * _NC + lax.axis_index("c")
        gsems = (g0, g1)
        osems = (o0, o1)

        def fire_gather(g, par, sem):
            idx_row0 = wid * idx_rows_per_w + g * _K_PER
            pltpu.sync_copy(idx_hbm.at[pl.ds(idx_row0, _K_PER)], idx_v.at[par])
            for j in range(_K_PER):
                pltpu.async_copy(
                    table_hbm.at[idx_v.at[par, j]],
                    rows_v.at[par, pl.ds(j * _IDX_MINOR, _IDX_MINOR)],
                    sem,
                )

        def drain_gather(par, sem):
            for j in range(_K_PER):
                pltpu.make_async_copy(
                    table_hbm.at[idx_v.at[par, j]],
                    rows_v.at[par, pl.ds(j * _IDX_MINOR, _IDX_MINOR)],
                    sem,
                ).wait()

        def wb_copy(g, par, sem):
            base = wid * b_per_w + g * _CHUNK
            return pltpu.make_async_copy(
                rows_v.at[par], out_hbm.at[pl.ds(base, _CHUNK)], sem
            )

        fire_gather(0, 0, gsems[0])
        fire_gather(1, 1, gsems[1])

        def loop_body(h, carry):
            for par in range(2):
                g = 2 * h + par
                drain_gather(par, gsems[par])
                wb_copy(g, par, osems[par]).start()

                @pl.when(g + 2 < n_chunks)
                def _(g=g, par=par):
                    wb_copy(g, par, osems[par]).wait()
                    fire_gather(g + 2, par, gsems[par])

            return carry

        lax.fori_loop(0, n_chunks // 2, loop_body, 0)
        wb_copy(n_chunks - 2, 0, osems[0]).wait()
        wb_copy(n_chunks - 1, 1, osems[1]).wait()

    return body


def kernel(x, emb_matrix):
    b, l = x.shape
    n = b * l
    idx2d = x.reshape(n // _IDX_MINOR, _IDX_MINOR).astype(jnp.int32)
    out = _make_gather(n)(idx2d, emb_matrix)
    return out.reshape(b, l, _EMB)


# speedup vs baseline: 1.0025x; 1.0025x over previous
import functools

import jax
import jax.numpy as jnp
from jax import lax
from jax.experimental import pallas as pl
from jax.experimental.pallas import tpu as pltpu
from jax.experimental.pallas import tpu_sc as plsc

_NC = 2
_NS = 16
_NW = _NC * _NS
_NB = 8


@functools.lru_cache(maxsize=None)
def _make_gather(b: int, l: int, emb: int):
    b_per_w = b // _NW
    n_chunks = b_per_w // _NB
    assert b_per_w * _NW == b and n_chunks * _NB == b_per_w
    assert n_chunks % 2 == 0
    mesh = plsc.VectorSubcoreMesh(core_axis_name="c", subcore_axis_name="s")

    @functools.partial(
        pl.kernel,
        out_type=jax.ShapeDtypeStruct((b, l, emb), jnp.float32),
        mesh=mesh,
        compiler_params=pltpu.CompilerParams(use_tc_tiling_on_sc=False),
        scratch_types=[
            pltpu.VMEM((2, _NB, l), jnp.int32),
            pltpu.VMEM((2, _NB, l, emb), jnp.float32),
            pltpu.SemaphoreType.DMA,
            pltpu.SemaphoreType.DMA,
            pltpu.SemaphoreType.DMA,
            pltpu.SemaphoreType.DMA,
        ],
    )
    def body(x_hbm, table_hbm, out_hbm, idx_v, rows_v, g0, g1, o0, o1):
        wid = lax.axis_index("s") * _NC + lax.axis_index("c")
        gsems = (g0, g1)
        osems = (o0, o1)

        def fire_gather(g, par, sem):
            b0 = wid * b_per_w + g * _NB
            pltpu.sync_copy(x_hbm.at[pl.ds(b0, _NB)], idx_v.at[par])
            for j in range(_NB):
                pltpu.async_copy(
                    table_hbm.at[idx_v.at[par, j]],
                    rows_v.at[par, j],
                    sem,
                )

        def drain_gather(par, sem):
            for j in range(_NB):
                pltpu.make_async_copy(
                    table_hbm.at[idx_v.at[par, j]],
                    rows_v.at[par, j],
                    sem,
                ).wait()

        def wb_copy(g, par, sem):
            b0 = wid * b_per_w + g * _NB
            return pltpu.make_async_copy(
                rows_v.at[par], out_hbm.at[pl.ds(b0, _NB)], sem
            )

        fire_gather(0, 0, gsems[0])
        fire_gather(1, 1, gsems[1])

        def loop_body(h, carry):
            for par in range(2):
                g = 2 * h + par
                drain_gather(par, gsems[par])
                wb_copy(g, par, osems[par]).start()

                @pl.when(g + 2 < n_chunks)
                def _(g=g, par=par):
                    wb_copy(g, par, osems[par]).wait()
                    fire_gather(g + 2, par, gsems[par])

            return carry

        lax.fori_loop(0, n_chunks // 2, loop_body, 0)
        wb_copy(n_chunks - 2, 0, osems[0]).wait()
        wb_copy(n_chunks - 1, 1, osems[1]).wait()

    return body


def kernel(x, emb_matrix):
    b, l = x.shape
    v, emb = emb_matrix.shape
    return _make_gather(b, l, emb)(x, emb_matrix)


# speedup vs baseline: 1.3493x; 1.3459x over previous
import functools

import jax
import jax.numpy as jnp
from jax import lax
from jax.experimental import pallas as pl
from jax.experimental.pallas import tpu as pltpu
from jax.experimental.pallas import tpu_sc as plsc

_NC = 2
_NS = 16
_NW = _NC * _NS
_CHUNK = 800


@functools.lru_cache(maxsize=None)
def _make_gather(n_tokens: int, emb: int):
    t_per_w = n_tokens // _NW
    n_chunks = t_per_w // _CHUNK
    assert t_per_w * _NW == n_tokens and n_chunks * _CHUNK == t_per_w
    assert n_chunks % 2 == 0
    mesh = plsc.VectorSubcoreMesh(core_axis_name="c", subcore_axis_name="s")

    @functools.partial(
        pl.kernel,
        out_type=jax.ShapeDtypeStruct((n_tokens, 128), jnp.float32),
        mesh=mesh,
        compiler_params=pltpu.CompilerParams(use_tc_tiling_on_sc=False),
        scratch_types=[
            pltpu.VMEM((2, _CHUNK), jnp.int32),
            pltpu.VMEM((2, _CHUNK, emb), jnp.float32),
            pltpu.SemaphoreType.DMA,
            pltpu.SemaphoreType.DMA,
            pltpu.SemaphoreType.DMA,
            pltpu.SemaphoreType.DMA,
        ],
    )
    def body(x_hbm, table_hbm, out_hbm, idx_v, rows_v, g0, g1, o0, o1):
        wid = lax.axis_index("s") * _NC + lax.axis_index("c")
        gsems = (g0, g1)
        osems = (o0, o1)

        def fire_gather(g, par, sem):
            row = wid * n_chunks + g
            pltpu.sync_copy(x_hbm.at[row], idx_v.at[par])
            pltpu.async_copy(
                table_hbm.at[idx_v.at[par]], rows_v.at[par], sem
            )

        def drain_gather(par, sem):
            pltpu.make_async_copy(
                table_hbm.at[idx_v.at[par]], rows_v.at[par], sem
            ).wait()

        def wb_copy(g, par, sem):
            t0 = (wid * n_chunks + g) * _CHUNK
            return pltpu.make_async_copy(
                rows_v.at[par],
                out_hbm.at[pl.ds(t0, _CHUNK), pl.ds(0, emb)],
                sem,
            )

        fire_gather(0, 0, gsems[0])
        fire_gather(1, 1, gsems[1])

        def loop_body(h, carry):
            for par in range(2):
                g = 2 * h + par
                drain_gather(par, gsems[par])
                wb_copy(g, par, osems[par]).start()

                @pl.when(g + 2 < n_chunks)
                def _(g=g, par=par):
                    wb_copy(g, par, osems[par]).wait()
                    fire_gather(g + 2, par, gsems[par])

            return carry

        lax.fori_loop(0, n_chunks // 2, loop_body, 0)
        wb_copy(n_chunks - 2, 0, osems[0]).wait()
        wb_copy(n_chunks - 1, 1, osems[1]).wait()

    return body


def kernel(x, emb_matrix):
    b, l = x.shape
    v, emb = emb_matrix.shape
    n = b * l
    x2d = x.reshape(n // _CHUNK, _CHUNK)
    table = lax.optimization_barrier(
        emb_matrix.reshape(v * emb // 128, 128)
    ).reshape(v, emb)
    out128 = _make_gather(n, emb)(x2d, table)
    return out128.reshape(b, l, 128)[:, :, :emb]


# speedup vs baseline: 1.3515x; 1.0016x over previous
import functools

import jax
import jax.numpy as jnp
from jax import lax
from jax.experimental import pallas as pl
from jax.experimental.pallas import tpu as pltpu
from jax.experimental.pallas import tpu_sc as plsc

_NC = 2
_NS = 16
_NW = _NC * _NS
_CHUNK = 800


@functools.lru_cache(maxsize=None)
def _make_gather(n_tokens: int, emb: int):
    t_per_w = n_tokens // _NW
    n_chunks = t_per_w // _CHUNK
    assert t_per_w * _NW == n_tokens and n_chunks * _CHUNK == t_per_w
    assert n_chunks % 2 == 0
    mesh = plsc.VectorSubcoreMesh(core_axis_name="c", subcore_axis_name="s")

    @functools.partial(
        pl.kernel,
        out_type=jax.ShapeDtypeStruct((n_tokens, 128), jnp.float32),
        mesh=mesh,
        compiler_params=pltpu.CompilerParams(use_tc_tiling_on_sc=False),
        scratch_types=[
            pltpu.VMEM((2, _CHUNK), jnp.int32),
            pltpu.VMEM((2, _CHUNK, emb), jnp.float32),
            pltpu.SemaphoreType.DMA,
            pltpu.SemaphoreType.DMA,
            pltpu.SemaphoreType.DMA,
            pltpu.SemaphoreType.DMA,
        ],
    )
    def body(x_hbm, table_hbm, out_hbm, idx_v, rows_v, g0, g1, o0, o1):
        wid = lax.axis_index("s") * _NC + lax.axis_index("c")
        gsems = (g0, g1)
        osems = (o0, o1)

        def fire_gather(g, par, sem):
            row = wid * n_chunks + g
            pltpu.sync_copy(x_hbm.at[row], idx_v.at[par])
            pltpu.async_copy(
                table_hbm.at[idx_v.at[par]], rows_v.at[par], sem
            )

        def drain_gather(par, sem):
            pltpu.make_async_copy(
                table_hbm.at[idx_v.at[par]], rows_v.at[par], sem
            ).wait()

        def wb_copy(g, par, sem):
            t0 = (wid * n_chunks + g) * _CHUNK
            return pltpu.make_async_copy(
                rows_v.at[par],
                out_hbm.at[pl.ds(t0, _CHUNK), pl.ds(0, emb)],
                sem,
            )

        fire_gather(0, 0, gsems[0])
        fire_gather(1, 1, gsems[1])

        def loop_body(h, carry):
            for par in range(2):
                g = 2 * h + par
                drain_gather(par, gsems[par])
                wb_copy(g, par, osems[par]).start()

                @pl.when(g + 2 < n_chunks)
                def _(g=g, par=par):
                    wb_copy(g, par, osems[par]).wait()
                    fire_gather(g + 2, par, gsems[par])

            return carry

        lax.fori_loop(0, n_chunks // 2, loop_body, 0)
        wb_copy(n_chunks - 2, 0, osems[0]).wait()
        wb_copy(n_chunks - 1, 1, osems[1]).wait()

    return body


def kernel(x, emb_matrix):
    b, l = x.shape
    v, emb = emb_matrix.shape
    n = b * l
    x2d = x.reshape(n // _CHUNK, _CHUNK)
    out128 = _make_gather(n, emb)(x2d, emb_matrix)
    return out128.reshape(b, l, 128)[:, :, :emb]


# speedup vs baseline: 1.3521x; 1.0005x over previous
import functools

import jax
import jax.numpy as jnp
from jax import lax
from jax.experimental import pallas as pl
from jax.experimental.pallas import tpu as pltpu
from jax.experimental.pallas import tpu_sc as plsc

_NC = 2
_NS = 16
_NW = _NC * _NS
_CHUNK = 800


@functools.lru_cache(maxsize=None)
def _make_gather(n_tokens: int, emb: int):
    t_per_w = n_tokens // _NW
    n_chunks = t_per_w // _CHUNK
    assert t_per_w * _NW == n_tokens and n_chunks * _CHUNK == t_per_w
    assert n_chunks % 2 == 0
    mesh = plsc.VectorSubcoreMesh(core_axis_name="c", subcore_axis_name="s")

    @functools.partial(
        pl.kernel,
        out_type=jax.ShapeDtypeStruct((n_tokens, 128), jnp.float32),
        mesh=mesh,
        compiler_params=pltpu.CompilerParams(use_tc_tiling_on_sc=False),
        scratch_types=[
            pltpu.VMEM((2, _CHUNK), jnp.int32),
            pltpu.VMEM((2, _CHUNK, emb), jnp.float32),
            pltpu.SemaphoreType.DMA,
            pltpu.SemaphoreType.DMA,
            pltpu.SemaphoreType.DMA,
            pltpu.SemaphoreType.DMA,
        ],
    )
    def body(x_hbm, table_hbm, out_hbm, idx_v, rows_v, g0, g1, o0, o1):
        wid = lax.axis_index("s") * _NC + lax.axis_index("c")
        gsems = (g0, g1)
        osems = (o0, o1)

        def fire_gather(g, par, sem):
            row = wid * n_chunks + g
            pltpu.sync_copy(x_hbm.at[row], idx_v.at[par])
            pltpu.async_copy(
                table_hbm.at[idx_v.at[par]], rows_v.at[par], sem
            )

        def drain_gather(par, sem):
            pltpu.make_async_copy(
                table_hbm.at[idx_v.at[par]], rows_v.at[par], sem
            ).wait()

        def wb_copy(g, par, sem):
            t0 = (wid * n_chunks + g) * _CHUNK
            return pltpu.make_async_copy(
                rows_v.at[par],
                out_hbm.at[pl.ds(t0, _CHUNK), pl.ds(0, emb)],
                sem,
            )

        fire_gather(0, 0, gsems[0])
        fire_gather(1, 1, gsems[1])

        def loop_body(h, carry):
            for par in range(2):
                g = 2 * h + par
                drain_gather(par, gsems[par])
                wb_copy(g, par, osems[par]).start()

                @pl.when(g + 2 < n_chunks)
                def _(g=g, par=par):
                    wb_copy(g, par, osems[par]).wait()
                    fire_gather(g + 2, par, gsems[par])

            return carry

        lax.fori_loop(0, n_chunks // 2, loop_body, 0)
        wb_copy(n_chunks - 2, 0, osems[0]).wait()
        wb_copy(n_chunks - 1, 1, osems[1]).wait()

    return body


def kernel(x, emb_matrix):
    b, l = x.shape
    v, emb = emb_matrix.shape
    n = b * l
    x2d = x.reshape(n // _CHUNK, _CHUNK)
    table = lax.optimization_barrier(emb_matrix.reshape(v * emb)).reshape(v, emb)
    out128 = _make_gather(n, emb)(x2d, table)
    return out128.reshape(b, l, 128)[:, :, :emb]
